# Initial kernel scaffold; baseline (speedup 1.0000x reference)
import jax
import jax.numpy as jnp
from jax.experimental import pallas as pl


def kernel(x, edge_index, W1, a1_src, a1_dst, b1, W2, a2_src, a2_dst, b2):
    raise NotImplementedError("write your pallas kernel here")


# speedup vs baseline: 5.9238x; 5.9238x over previous
import functools

import jax
import jax.numpy as jnp
from jax import lax
from jax.experimental import pallas as pl
from jax.experimental.pallas import tpu as pltpu
from jax.experimental.pallas import tpu_sc as plsc

NC = 2
NS = 16
L = 16

N = 10000
NP = 10240
E = 320000
EPAD = 327680
H = 8
D = 128

MC = 1024
SB = 64
_EW = EPAD // NS
_RS = NP // NS
_WP = EPAD // 8
_WR = EPAD // 128


def _proj_common(h, As_ref, Ad_ref, hT_ref, AA_ref, M_ref, mA_ref, mB_ref):
    i = pl.program_id(0)
    for hh in range(H):
        hT_ref[hh] = h[:, D * hh:D * (hh + 1)]
    AS_blk = jnp.dot(h, As_ref[...], preferred_element_type=jnp.float32)
    AD_blk = jnp.dot(h, Ad_ref[...], preferred_element_type=jnp.float32)
    AA_ref[...] = jnp.concatenate(
        [AS_blk, AD_blk, jnp.zeros((AS_blk.shape[0], 96), jnp.float32)],
        axis=1)

    @pl.when(i == 0)
    def _():
        mA_ref[...] = jnp.full((8, 16), -1e30, jnp.float32)
        mB_ref[...] = jnp.full((8, 16), -1e30, jnp.float32)

    mA = jnp.max(AS_blk, axis=0, keepdims=True)
    mB = jnp.max(AD_blk, axis=0, keepdims=True)
    mA_ref[...] = jnp.maximum(mA_ref[...], jnp.broadcast_to(mA, (8, 16)))
    mB_ref[...] = jnp.maximum(mB_ref[...], jnp.broadcast_to(mB, (8, 16)))
    M_ref[...] = jnp.maximum(mA_ref[...] + mB_ref[...], 0.0)


def _dense1_body(x_ref, W_ref, As_ref, Ad_ref, hT_ref, AA_ref, M_ref,
                 mA_ref, mB_ref):
    h = jnp.dot(x_ref[...], W_ref[...], preferred_element_type=jnp.float32)
    _proj_common(h, As_ref, Ad_ref, hT_ref, AA_ref, M_ref, mA_ref, mB_ref)


def _dense2_body(in_ref, b_ref, W_ref, As_ref, Ad_ref, hT_ref, AA_ref, M_ref,
                 mA_ref, mB_ref):
    acc = in_ref[0]
    for hh in range(1, H):
        acc = acc + in_ref[hh]
    x2 = jnp.maximum(acc * (1.0 / H) + b_ref[...], 0.0)
    h = jnp.dot(x2, W_ref[...], preferred_element_type=jnp.float32)
    _proj_common(h, As_ref, Ad_ref, hT_ref, AA_ref, M_ref, mA_ref, mB_ref)


def _final_body(in_ref, b_ref, out_ref):
    acc = in_ref[0]
    for hh in range(1, H):
        acc = acc + in_ref[hh]
    y = acc * (1.0 / H) + b_ref[...]
    m = jnp.max(y, axis=1, keepdims=True)
    lse = jnp.log(jnp.sum(jnp.exp(y - m), axis=1, keepdims=True)) + m
    out_ref[...] = y - lse


_BP = 640
_BF = 1000

_PROJ_OUT_SPECS = [
    pl.BlockSpec((H, _BP, D), lambda i: (0, i, 0)),
    pl.BlockSpec((_BP, D), lambda i: (i, 0)),
    pl.BlockSpec((8, 16), lambda i: (0, 0)),
]
_PROJ_OUT_SHAPE = [
    jax.ShapeDtypeStruct((H, NP, D), jnp.float32),
    jax.ShapeDtypeStruct((NP, D), jnp.float32),
    jax.ShapeDtypeStruct((8, 16), jnp.float32),
]
_PROJ_SCRATCH = [
    pltpu.VMEM((8, 16), jnp.float32),
    pltpu.VMEM((8, 16), jnp.float32),
]

_dense1 = pl.pallas_call(
    _dense1_body,
    grid=(NP // _BP,),
    in_specs=[
        pl.BlockSpec((_BP, D), lambda i: (i, 0)),
        pl.BlockSpec((D, H * D), lambda i: (0, 0)),
        pl.BlockSpec((H * D, 16), lambda i: (0, 0)),
        pl.BlockSpec((H * D, 16), lambda i: (0, 0)),
    ],
    out_specs=_PROJ_OUT_SPECS,
    out_shape=_PROJ_OUT_SHAPE,
    scratch_shapes=_PROJ_SCRATCH,
)

_dense2 = pl.pallas_call(
    _dense2_body,
    grid=(NP // _BP,),
    in_specs=[
        pl.BlockSpec((H, _BP, D), lambda i: (0, i, 0)),
        pl.BlockSpec((1, D), lambda i: (0, 0)),
        pl.BlockSpec((D, H * D), lambda i: (0, 0)),
        pl.BlockSpec((H * D, 16), lambda i: (0, 0)),
        pl.BlockSpec((H * D, 16), lambda i: (0, 0)),
    ],
    out_specs=_PROJ_OUT_SPECS,
    out_shape=_PROJ_OUT_SHAPE,
    scratch_shapes=_PROJ_SCRATCH,
)

_final = pl.pallas_call(
    _final_body,
    grid=(N // _BF,),
    in_specs=[
        pl.BlockSpec((H, _BF, D), lambda i: (0, i, 0)),
        pl.BlockSpec((1, D), lambda i: (0, 0)),
    ],
    out_specs=pl.BlockSpec((_BF, D), lambda i: (i, 0)),
    out_shape=jax.ShapeDtypeStruct((N, D), jnp.float32),
)


def _edge_w_body(src_hbm, dst_hbm, AA_hbm, Mp_hbm, w_hbm, al_hbm, den_hbm,
                 src8, dst8, ixs, ixd, asg, adg, dgath, wb, apb, apb2, dnb,
                 dwb, mvb, sem, den_sh):
    c = lax.axis_index("c")
    s = lax.axis_index("s")

    def zrow(r, _):
        dnb[r, :] = jnp.zeros((L,), jnp.float32)
        return _

    lax.fori_loop(0, 64, zrow, None)
    for g in range(_RS // 64):
        pltpu.sync_copy(dnb, den_sh.at[pl.ds(s * _RS + 64 * g, 64)])

    for r in range(64):
        for j in range(1, D // L):
            dwb[r, pl.ds(L * j, L)] = jnp.zeros((L,), jnp.float32)

    pltpu.sync_copy(Mp_hbm, mvb)
    mv = mvb[0, pl.ds(0, L)]
    plsc.subcore_barrier()

    def mchunk1(ci, _):
        e0 = s * _EW + ci * MC
        r0 = pl.multiple_of(e0 // 128, 8)
        pltpu.sync_copy(src_hbm.at[pl.ds(r0, MC // 128)], src8)
        pltpu.sync_copy(dst_hbm.at[pl.ds(r0, MC // 128)], dst8)
        for sub in range(MC // SB):
            for t in range(SB // L):
                ixs[sub, pl.ds(L * t, L)] = src8[
                    sub // 2, pl.ds((sub % 2) * SB + L * t, L)]
                ixd[sub, pl.ds(L * t, L)] = dst8[
                    sub // 2, pl.ds((sub % 2) * SB + L * t, L)]

        def subfn(sub, _):
            d1 = pltpu.async_copy(AA_hbm.at[ixs.at[sub]], asg, sem)
            d2 = pltpu.async_copy(AA_hbm.at[ixd.at[sub]], adg, sem)
            d1.wait()
            d2.wait()
            for i in range(SB):
                z = asg[i, pl.ds(0, L)] + adg[i, pl.ds(L, L)]
                z = jnp.where(z < 0, z * 0.2, z) - mv
                w = jnp.exp(z)
                wb[i, :] = w
                apb[i // 8, pl.ds(L * (i % 8), L)] = w
            pltpu.sync_copy(wb, den_sh.at[ixd.at[sub]], add=True)
            aoff = pl.multiple_of(e0 // 8 + sub * (SB // 8), 8)
            pltpu.sync_copy(apb, w_hbm.at[pl.ds(aoff, SB // 8)])
            return _

        lax.fori_loop(0, MC // SB, subfn, None)
        return _

    lax.fori_loop(0, _EW // MC, mchunk1, None)
    plsc.subcore_barrier()

    def dwrite(g, _):
        pltpu.sync_copy(den_sh.at[pl.ds(s * _RS + 64 * g, 64)], dnb)
        for r in range(64):
            dwb[r, pl.ds(0, L)] = dnb[r, :]
        doff = pl.multiple_of(s * _RS + 64 * g, 8)
        pltpu.sync_copy(dwb, den_hbm.at[pl.ds(doff, 64)])
        return _

    lax.fori_loop(0, _RS // 64, dwrite, None)
    plsc.subcore_barrier()

    def mchunk2(ci, _):
        e0 = s * _EW + ci * MC
        r0 = pl.multiple_of(e0 // 128, 8)
        pltpu.sync_copy(dst_hbm.at[pl.ds(r0, MC // 128)], dst8)
        for sub in range(MC // SB):
            for t in range(SB // L):
                ixd[sub, pl.ds(L * t, L)] = dst8[
                    sub // 2, pl.ds((sub % 2) * SB + L * t, L)]

        def subfn(sub, _):
            pltpu.async_copy(den_hbm.at[ixd.at[sub]], dgath, sem).wait()
            aoff = pl.multiple_of(e0 // 8 + sub * (SB // 8), 8)
            pltpu.sync_copy(w_hbm.at[pl.ds(aoff, SB // 8)], apb)
            for i in range(SB):
                wseg = apb[i // 8, pl.ds(L * (i % 8), L)]
                dseg = dgath[i, pl.ds(0, L)]
                apb2[i // 8, pl.ds(L * (i % 8), L)] = (
                    wseg / (dseg + 1e-16))
            pltpu.sync_copy(apb2, al_hbm.at[pl.ds(aoff, SB // 8)])
            return _

        lax.fori_loop(0, MC // SB, subfn, None)
        return _

    lax.fori_loop(0, _EW // MC, mchunk2, None)


def _msg_body(src_hbm, dst_hbm, alT_hbm, h2d_hbm, out_hbm,
              src8, dst8, ixs, ixd, rows, wvb, obuf, sem, acc_sh):
    c = lax.axis_index("c")
    s = lax.axis_index("s")

    def headfn(hh, _):
        k = c * (H // NC) + hh
        kNp = k * NP
        kwr = k * _WR

        for r in range(64):
            for j in range(D // L):
                obuf[r, pl.ds(L * j, L)] = jnp.zeros((L,), jnp.float32)
        for g in range(_RS // 64):
            pltpu.sync_copy(obuf, acc_sh.at[pl.ds(s * _RS + 64 * g, 64)])
        plsc.subcore_barrier()

        def mchunk(ci, _):
            e0 = s * _EW + ci * MC
            r0 = pl.multiple_of(e0 // 128, 8)
            pltpu.sync_copy(src_hbm.at[pl.ds(r0, MC // 128)], src8)
            pltpu.sync_copy(dst_hbm.at[pl.ds(r0, MC // 128)], dst8)
            woff = pl.multiple_of(kwr + e0 // 128, 8)
            pltpu.sync_copy(alT_hbm.at[pl.ds(woff, MC // 128)], wvb)
            for sub in range(MC // SB):
                for t in range(SB // L):
                    ixs[sub, pl.ds(L * t, L)] = (
                        src8[sub // 2, pl.ds((sub % 2) * SB + L * t, L)]
                        + kNp)
                    ixd[sub, pl.ds(L * t, L)] = dst8[
                        sub // 2, pl.ds((sub % 2) * SB + L * t, L)]

            def subfn(sub, _):
                pltpu.async_copy(h2d_hbm.at[ixs.at[sub]], rows, sem).wait()
                for i in range(SB):
                    aseg = wvb[sub // 2,
                               pl.ds((sub % 2) * SB + L * (i // L), L)]
                    av = jnp.full((L,), aseg[i % L], jnp.float32)
                    for j in range(D // L):
                        rows[i, pl.ds(L * j, L)] = (
                            rows[i, pl.ds(L * j, L)] * av)
                pltpu.sync_copy(rows, acc_sh.at[ixd.at[sub]], add=True)
                return _

            lax.fori_loop(0, MC // SB, subfn, None)
            return _

        lax.fori_loop(0, _EW // MC, mchunk, None)
        plsc.subcore_barrier()

        def wout(g, _):
            pltpu.sync_copy(acc_sh.at[pl.ds(s * _RS + 64 * g, 64)], obuf)
            ooff = pl.multiple_of(kNp + s * _RS + 64 * g, 8)
            pltpu.sync_copy(obuf, out_hbm.at[pl.ds(ooff, 64)])
            return _

        lax.fori_loop(0, _RS // 64, wout, None)
        plsc.subcore_barrier()
        return _

    lax.fori_loop(0, H // NC, headfn, None)


def _make_sc_kernels():
    mesh = plsc.VectorSubcoreMesh(core_axis_name="c", subcore_axis_name="s",
                                  num_cores=NC, num_subcores=NS)
    edge_w = functools.partial(
        pl.kernel,
        mesh=mesh,
        compiler_params=pltpu.CompilerParams(use_tc_tiling_on_sc=False),
        out_type=[
            jax.ShapeDtypeStruct((_WP, D), jnp.float32),
            jax.ShapeDtypeStruct((_WP, D), jnp.float32),
            jax.ShapeDtypeStruct((NP, D), jnp.float32),
        ],
        scratch_types=[
            pltpu.VMEM((MC // 128, 128), jnp.int32),
            pltpu.VMEM((MC // 128, 128), jnp.int32),
            pltpu.VMEM((MC // SB, SB), jnp.int32),
            pltpu.VMEM((MC // SB, SB), jnp.int32),
            pltpu.VMEM((SB, D), jnp.float32),
            pltpu.VMEM((SB, D), jnp.float32),
            pltpu.VMEM((SB, D), jnp.float32),
            pltpu.VMEM((SB, L), jnp.float32),
            pltpu.VMEM((SB // 8, D), jnp.float32),
            pltpu.VMEM((SB // 8, D), jnp.float32),
            pltpu.VMEM((64, L), jnp.float32),
            pltpu.VMEM((64, D), jnp.float32),
            pltpu.VMEM((8, D), jnp.float32),
            pltpu.SemaphoreType.DMA,
            pltpu.VMEM_SHARED((NP, L), jnp.float32),
        ],
    )(_edge_w_body)
    msg = functools.partial(
        pl.kernel,
        mesh=mesh,
        compiler_params=pltpu.CompilerParams(use_tc_tiling_on_sc=False),
        out_type=jax.ShapeDtypeStruct((H * NP, D), jnp.float32),
        scratch_types=[
            pltpu.VMEM((MC // 128, 128), jnp.int32),
            pltpu.VMEM((MC // 128, 128), jnp.int32),
            pltpu.VMEM((MC // SB, SB), jnp.int32),
            pltpu.VMEM((MC // SB, SB), jnp.int32),
            pltpu.VMEM((SB, D), jnp.float32),
            pltpu.VMEM((MC // 128, 128), jnp.float32),
            pltpu.VMEM((64, D), jnp.float32),
            pltpu.SemaphoreType.DMA,
            pltpu.VMEM_SHARED((NP, D), jnp.float32),
        ],
    )(_msg_body)
    return edge_w, msg


_SC_KERNELS = None


def _attn_mats(a_src, a_dst):
    eye = jnp.eye(H, 16, dtype=jnp.float32)
    AsM = (a_src[:, :, None] * eye[:, None, :]).reshape(H * D, 16)
    AdM = (a_dst[:, :, None] * eye[:, None, :]).reshape(H * D, 16)
    return AsM, AdM


def _per_head_alpha(al):
    arr = al.reshape(_WP, 8, 16)[:, :, :H]
    return arr.transpose(2, 0, 1).reshape(H * _WR, D)


def kernel(x, edge_index, W1, a1_src, a1_dst, b1, W2, a2_src, a2_dst, b2):
    global _SC_KERNELS
    if _SC_KERNELS is None:
        _SC_KERNELS = _make_sc_kernels()
    edge_w, msg = _SC_KERNELS

    src = edge_index[0]
    dst = edge_index[1]
    padn = jnp.full((EPAD - E,), N, jnp.int32)
    srcp = jnp.concatenate([src, padn]).reshape(EPAD // 128, 128)
    dstp = jnp.concatenate([dst, padn]).reshape(EPAD // 128, 128)
    x_pad = jnp.concatenate([x, jnp.zeros((NP - N, D), jnp.float32)])

    As1M, Ad1M = _attn_mats(a1_src, a1_dst)
    As2M, Ad2M = _attn_mats(a2_src, a2_dst)

    hT1, AA1, M1 = _dense1(x_pad, W1, As1M, Ad1M)
    _, al1, _ = edge_w(srcp, dstp, AA1, jnp.tile(M1.reshape(1, D), (8, 1)))
    out1 = msg(srcp, dstp, _per_head_alpha(al1), hT1.reshape(H * NP, D))

    hT2, AA2, M2 = _dense2(out1.reshape(H, NP, D), b1.reshape(1, D), W2,
                           As2M, Ad2M)
    _, al2, _ = edge_w(srcp, dstp, AA2, jnp.tile(M2.reshape(1, D), (8, 1)))
    out2 = msg(srcp, dstp, _per_head_alpha(al2), hT2.reshape(H * NP, D))

    return _final(out2.reshape(H, NP, D), b2.reshape(1, D))


# speedup vs baseline: 5.9409x; 1.0029x over previous
import functools

import jax
import jax.numpy as jnp
from jax import lax
from jax.experimental import pallas as pl
from jax.experimental.pallas import tpu as pltpu
from jax.experimental.pallas import tpu_sc as plsc

NC = 2
NS = 16
L = 16

N = 10000
NP = 10240
E = 320000
EPAD = 327680
H = 8
D = 128

MC = 1024
SB = 128
_EW = EPAD // NS
_RS = NP // NS
_WP = EPAD // 8
_WR = EPAD // 128


def _proj_common(h, As_ref, Ad_ref, hT_ref, AA_ref, M_ref, mA_ref, mB_ref):
    i = pl.program_id(0)
    for hh in range(H):
        hT_ref[hh] = h[:, D * hh:D * (hh + 1)]
    AS_blk = jnp.dot(h, As_ref[...], preferred_element_type=jnp.float32)
    AD_blk = jnp.dot(h, Ad_ref[...], preferred_element_type=jnp.float32)
    AA_ref[...] = jnp.concatenate(
        [AS_blk, AD_blk, jnp.zeros((AS_blk.shape[0], 96), jnp.float32)],
        axis=1)

    @pl.when(i == 0)
    def _():
        mA_ref[...] = jnp.full((8, 16), -1e30, jnp.float32)
        mB_ref[...] = jnp.full((8, 16), -1e30, jnp.float32)

    mA = jnp.max(AS_blk, axis=0, keepdims=True)
    mB = jnp.max(AD_blk, axis=0, keepdims=True)
    mA_ref[...] = jnp.maximum(mA_ref[...], jnp.broadcast_to(mA, (8, 16)))
    mB_ref[...] = jnp.maximum(mB_ref[...], jnp.broadcast_to(mB, (8, 16)))
    M_ref[...] = jnp.maximum(mA_ref[...] + mB_ref[...], 0.0)


def _dense1_body(x_ref, W_ref, As_ref, Ad_ref, hT_ref, AA_ref, M_ref,
                 mA_ref, mB_ref):
    h = jnp.dot(x_ref[...], W_ref[...], preferred_element_type=jnp.float32)
    _proj_common(h, As_ref, Ad_ref, hT_ref, AA_ref, M_ref, mA_ref, mB_ref)


def _dense2_body(in_ref, b_ref, W_ref, As_ref, Ad_ref, hT_ref, AA_ref, M_ref,
                 mA_ref, mB_ref):
    acc = in_ref[0]
    for hh in range(1, H):
        acc = acc + in_ref[hh]
    x2 = jnp.maximum(acc * (1.0 / H) + b_ref[...], 0.0)
    h = jnp.dot(x2, W_ref[...], preferred_element_type=jnp.float32)
    _proj_common(h, As_ref, Ad_ref, hT_ref, AA_ref, M_ref, mA_ref, mB_ref)


def _final_body(in_ref, b_ref, out_ref):
    acc = in_ref[0]
    for hh in range(1, H):
        acc = acc + in_ref[hh]
    y = acc * (1.0 / H) + b_ref[...]
    m = jnp.max(y, axis=1, keepdims=True)
    lse = jnp.log(jnp.sum(jnp.exp(y - m), axis=1, keepdims=True)) + m
    out_ref[...] = y - lse


_BP = 640
_BF = 1000

_PROJ_OUT_SPECS = [
    pl.BlockSpec((H, _BP, D), lambda i: (0, i, 0)),
    pl.BlockSpec((_BP, D), lambda i: (i, 0)),
    pl.BlockSpec((8, 16), lambda i: (0, 0)),
]
_PROJ_OUT_SHAPE = [
    jax.ShapeDtypeStruct((H, NP, D), jnp.float32),
    jax.ShapeDtypeStruct((NP, D), jnp.float32),
    jax.ShapeDtypeStruct((8, 16), jnp.float32),
]
_PROJ_SCRATCH = [
    pltpu.VMEM((8, 16), jnp.float32),
    pltpu.VMEM((8, 16), jnp.float32),
]

_dense1 = pl.pallas_call(
    _dense1_body,
    grid=(NP // _BP,),
    in_specs=[
        pl.BlockSpec((_BP, D), lambda i: (i, 0)),
        pl.BlockSpec((D, H * D), lambda i: (0, 0)),
        pl.BlockSpec((H * D, 16), lambda i: (0, 0)),
        pl.BlockSpec((H * D, 16), lambda i: (0, 0)),
    ],
    out_specs=_PROJ_OUT_SPECS,
    out_shape=_PROJ_OUT_SHAPE,
    scratch_shapes=_PROJ_SCRATCH,
)

_dense2 = pl.pallas_call(
    _dense2_body,
    grid=(NP // _BP,),
    in_specs=[
        pl.BlockSpec((H, _BP, D), lambda i: (0, i, 0)),
        pl.BlockSpec((1, D), lambda i: (0, 0)),
        pl.BlockSpec((D, H * D), lambda i: (0, 0)),
        pl.BlockSpec((H * D, 16), lambda i: (0, 0)),
        pl.BlockSpec((H * D, 16), lambda i: (0, 0)),
    ],
    out_specs=_PROJ_OUT_SPECS,
    out_shape=_PROJ_OUT_SHAPE,
    scratch_shapes=_PROJ_SCRATCH,
)

_final = pl.pallas_call(
    _final_body,
    grid=(N // _BF,),
    in_specs=[
        pl.BlockSpec((H, _BF, D), lambda i: (0, i, 0)),
        pl.BlockSpec((1, D), lambda i: (0, 0)),
    ],
    out_specs=pl.BlockSpec((_BF, D), lambda i: (i, 0)),
    out_shape=jax.ShapeDtypeStruct((N, D), jnp.float32),
)


def _edge_w_body(src_hbm, dst_hbm, AA_hbm, Mp_hbm, w_hbm, al_hbm, den_hbm,
                 src8, dst8, ixs, ixd, asg, adg, dgath, wb, apb, apb2, dnb,
                 dwb, mvb, sem, den_sh):
    c = lax.axis_index("c")
    s = lax.axis_index("s")

    def zrow(r, _):
        dnb[r, :] = jnp.zeros((L,), jnp.float32)
        return _

    lax.fori_loop(0, 64, zrow, None)
    for g in range(_RS // 64):
        pltpu.sync_copy(dnb, den_sh.at[pl.ds(s * _RS + 64 * g, 64)])

    for r in range(64):
        for j in range(1, D // L):
            dwb[r, pl.ds(L * j, L)] = jnp.zeros((L,), jnp.float32)

    pltpu.sync_copy(Mp_hbm, mvb)
    mv = mvb[0, pl.ds(0, L)]
    plsc.subcore_barrier()

    def mchunk1(ci, _):
        e0 = s * _EW + ci * MC
        r0 = pl.multiple_of(e0 // 128, 8)
        pltpu.sync_copy(src_hbm.at[pl.ds(r0, MC // 128)], src8)
        pltpu.sync_copy(dst_hbm.at[pl.ds(r0, MC // 128)], dst8)
        for sub in range(MC // SB):
            for t in range(SB // L):
                ixs[sub, pl.ds(L * t, L)] = src8[sub, pl.ds(L * t, L)]
                ixd[sub, pl.ds(L * t, L)] = dst8[sub, pl.ds(L * t, L)]

        def subfn(sub, _):
            d1 = pltpu.async_copy(AA_hbm.at[ixs.at[sub]], asg, sem)
            d2 = pltpu.async_copy(AA_hbm.at[ixd.at[sub]], adg, sem)
            d1.wait()
            d2.wait()
            for i in range(SB):
                z = asg[i, pl.ds(0, L)] + adg[i, pl.ds(L, L)]
                z = jnp.where(z < 0, z * 0.2, z) - mv
                w = jnp.exp(z)
                wb[i, :] = w
                apb[i // 8, pl.ds(L * (i % 8), L)] = w
            pltpu.sync_copy(wb, den_sh.at[ixd.at[sub]], add=True)
            aoff = pl.multiple_of(e0 // 8 + sub * (SB // 8), 8)
            pltpu.sync_copy(apb, w_hbm.at[pl.ds(aoff, SB // 8)])
            return _

        lax.fori_loop(0, MC // SB, subfn, None)
        return _

    lax.fori_loop(0, _EW // MC, mchunk1, None)
    plsc.subcore_barrier()

    def dwrite(g, _):
        pltpu.sync_copy(den_sh.at[pl.ds(s * _RS + 64 * g, 64)], dnb)
        for r in range(64):
            dwb[r, pl.ds(0, L)] = dnb[r, :]
        doff = pl.multiple_of(s * _RS + 64 * g, 8)
        pltpu.sync_copy(dwb, den_hbm.at[pl.ds(doff, 64)])
        return _

    lax.fori_loop(0, _RS // 64, dwrite, None)
    plsc.subcore_barrier()

    def mchunk2(ci, _):
        e0 = s * _EW + ci * MC
        r0 = pl.multiple_of(e0 // 128, 8)
        pltpu.sync_copy(dst_hbm.at[pl.ds(r0, MC // 128)], dst8)
        for sub in range(MC // SB):
            for t in range(SB // L):
                ixd[sub, pl.ds(L * t, L)] = dst8[sub, pl.ds(L * t, L)]

        def subfn(sub, _):
            pltpu.async_copy(den_hbm.at[ixd.at[sub]], dgath, sem).wait()
            aoff = pl.multiple_of(e0 // 8 + sub * (SB // 8), 8)
            pltpu.sync_copy(w_hbm.at[pl.ds(aoff, SB // 8)], apb)
            for i in range(SB):
                wseg = apb[i // 8, pl.ds(L * (i % 8), L)]
                dseg = dgath[i, pl.ds(0, L)]
                apb2[i // 8, pl.ds(L * (i % 8), L)] = (
                    wseg / (dseg + 1e-16))
            pltpu.sync_copy(apb2, al_hbm.at[pl.ds(aoff, SB // 8)])
            return _

        lax.fori_loop(0, MC // SB, subfn, None)
        return _

    lax.fori_loop(0, _EW // MC, mchunk2, None)


def _msg_body(src_hbm, dst_hbm, alT_hbm, h2d_hbm, out_hbm,
              src8, dst8, ixs, ixd, rows, wvb, obuf, sem, acc_sh):
    c = lax.axis_index("c")
    s = lax.axis_index("s")

    def headfn(hh, _):
        k = c * (H // NC) + hh
        kNp = k * NP
        kwr = k * _WR

        for r in range(64):
            for j in range(D // L):
                obuf[r, pl.ds(L * j, L)] = jnp.zeros((L,), jnp.float32)
        for g in range(_RS // 64):
            pltpu.sync_copy(obuf, acc_sh.at[pl.ds(s * _RS + 64 * g, 64)])
        plsc.subcore_barrier()

        def mchunk(ci, _):
            e0 = s * _EW + ci * MC
            r0 = pl.multiple_of(e0 // 128, 8)
            pltpu.sync_copy(src_hbm.at[pl.ds(r0, MC // 128)], src8)
            pltpu.sync_copy(dst_hbm.at[pl.ds(r0, MC // 128)], dst8)
            woff = pl.multiple_of(kwr + e0 // 128, 8)
            pltpu.sync_copy(alT_hbm.at[pl.ds(woff, MC // 128)], wvb)
            for sub in range(MC // SB):
                for t in range(SB // L):
                    ixs[sub, pl.ds(L * t, L)] = (
                        src8[sub, pl.ds(L * t, L)] + kNp)
                    ixd[sub, pl.ds(L * t, L)] = dst8[sub, pl.ds(L * t, L)]

            def subfn(sub, _):
                pltpu.async_copy(h2d_hbm.at[ixs.at[sub]], rows, sem).wait()
                for i in range(SB):
                    aseg = wvb[sub, pl.ds(L * (i // L), L)]
                    av = jnp.full((L,), aseg[i % L], jnp.float32)
                    for j in range(D // L):
                        rows[i, pl.ds(L * j, L)] = (
                            rows[i, pl.ds(L * j, L)] * av)
                pltpu.sync_copy(rows, acc_sh.at[ixd.at[sub]], add=True)
                return _

            lax.fori_loop(0, MC // SB, subfn, None)
            return _

        lax.fori_loop(0, _EW // MC, mchunk, None)
        plsc.subcore_barrier()

        def wout(g, _):
            pltpu.sync_copy(acc_sh.at[pl.ds(s * _RS + 64 * g, 64)], obuf)
            ooff = pl.multiple_of(kNp + s * _RS + 64 * g, 8)
            pltpu.sync_copy(obuf, out_hbm.at[pl.ds(ooff, 64)])
            return _

        lax.fori_loop(0, _RS // 64, wout, None)
        plsc.subcore_barrier()
        return _

    lax.fori_loop(0, H // NC, headfn, None)


def _make_sc_kernels():
    mesh = plsc.VectorSubcoreMesh(core_axis_name="c", subcore_axis_name="s",
                                  num_cores=NC, num_subcores=NS)
    edge_w = functools.partial(
        pl.kernel,
        mesh=mesh,
        compiler_params=pltpu.CompilerParams(use_tc_tiling_on_sc=False),
        out_type=[
            jax.ShapeDtypeStruct((_WP, D), jnp.float32),
            jax.ShapeDtypeStruct((_WP, D), jnp.float32),
            jax.ShapeDtypeStruct((NP, D), jnp.float32),
        ],
        scratch_types=[
            pltpu.VMEM((MC // 128, 128), jnp.int32),
            pltpu.VMEM((MC // 128, 128), jnp.int32),
            pltpu.VMEM((MC // SB, SB), jnp.int32),
            pltpu.VMEM((MC // SB, SB), jnp.int32),
            pltpu.VMEM((SB, D), jnp.float32),
            pltpu.VMEM((SB, D), jnp.float32),
            pltpu.VMEM((SB, D), jnp.float32),
            pltpu.VMEM((SB, L), jnp.float32),
            pltpu.VMEM((SB // 8, D), jnp.float32),
            pltpu.VMEM((SB // 8, D), jnp.float32),
            pltpu.VMEM((64, L), jnp.float32),
            pltpu.VMEM((64, D), jnp.float32),
            pltpu.VMEM((8, D), jnp.float32),
            pltpu.SemaphoreType.DMA,
            pltpu.VMEM_SHARED((NP, L), jnp.float32),
        ],
    )(_edge_w_body)
    msg = functools.partial(
        pl.kernel,
        mesh=mesh,
        compiler_params=pltpu.CompilerParams(use_tc_tiling_on_sc=False),
        out_type=jax.ShapeDtypeStruct((H * NP, D), jnp.float32),
        scratch_types=[
            pltpu.VMEM((MC // 128, 128), jnp.int32),
            pltpu.VMEM((MC // 128, 128), jnp.int32),
            pltpu.VMEM((MC // SB, SB), jnp.int32),
            pltpu.VMEM((MC // SB, SB), jnp.int32),
            pltpu.VMEM((SB, D), jnp.float32),
            pltpu.VMEM((MC // 128, 128), jnp.float32),
            pltpu.VMEM((64, D), jnp.float32),
            pltpu.SemaphoreType.DMA,
            pltpu.VMEM_SHARED((NP, D), jnp.float32),
        ],
    )(_msg_body)
    return edge_w, msg


_SC_KERNELS = None


def _attn_mats(a_src, a_dst):
    eye = jnp.eye(H, 16, dtype=jnp.float32)
    AsM = (a_src[:, :, None] * eye[:, None, :]).reshape(H * D, 16)
    AdM = (a_dst[:, :, None] * eye[:, None, :]).reshape(H * D, 16)
    return AsM, AdM


def _per_head_alpha(al):
    arr = al.reshape(_WP, 8, 16)[:, :, :H]
    return arr.transpose(2, 0, 1).reshape(H * _WR, D)


def kernel(x, edge_index, W1, a1_src, a1_dst, b1, W2, a2_src, a2_dst, b2):
    global _SC_KERNELS
    if _SC_KERNELS is None:
        _SC_KERNELS = _make_sc_kernels()
    edge_w, msg = _SC_KERNELS

    src = edge_index[0]
    dst = edge_index[1]
    padn = jnp.full((EPAD - E,), N, jnp.int32)
    srcp = jnp.concatenate([src, padn]).reshape(EPAD // 128, 128)
    dstp = jnp.concatenate([dst, padn]).reshape(EPAD // 128, 128)
    x_pad = jnp.concatenate([x, jnp.zeros((NP - N, D), jnp.float32)])

    As1M, Ad1M = _attn_mats(a1_src, a1_dst)
    As2M, Ad2M = _attn_mats(a2_src, a2_dst)

    hT1, AA1, M1 = _dense1(x_pad, W1, As1M, Ad1M)
    _, al1, _ = edge_w(srcp, dstp, AA1, jnp.tile(M1.reshape(1, D), (8, 1)))
    out1 = msg(srcp, dstp, _per_head_alpha(al1), hT1.reshape(H * NP, D))

    hT2, AA2, M2 = _dense2(out1.reshape(H, NP, D), b1.reshape(1, D), W2,
                           As2M, Ad2M)
    _, al2, _ = edge_w(srcp, dstp, AA2, jnp.tile(M2.reshape(1, D), (8, 1)))
    out2 = msg(srcp, dstp, _per_head_alpha(al2), hT2.reshape(H * NP, D))

    return _final(out2.reshape(H, NP, D), b2.reshape(1, D))
